# baseline (speedup 1.0000x reference)
import jax
import jax.numpy as jnp
from jax.experimental import pallas as pl
from jax.experimental.pallas import tpu as pltpu

_C = 4
_K = 4


def _scale_body(stage_ref, emb_ref, x_hbm, o_hbm, in_buf, out_buf,
                in_sem, out_sem):
    n = x_hbm.shape[0] // _C
    s = stage_ref[0]
    emb = emb_ref[...]
    row = jax.lax.broadcasted_iota(jnp.int32, emb.shape, 0)
    scale = jnp.sum(jnp.where(row == s, emb, 0.0), axis=0)

    def in_copy(i, slot):
        return pltpu.make_async_copy(x_hbm.at[pl.ds(i * _C, _C)],
                                     in_buf.at[slot], in_sem.at[slot])

    def out_copy(i, slot):
        return pltpu.make_async_copy(out_buf.at[slot],
                                     o_hbm.at[pl.ds(i * _C, _C)],
                                     out_sem.at[slot])

    for k in range(_K):
        in_copy(k, k).start()

    def step(i, carry):
        slot = jax.lax.rem(i, _K)
        in_copy(i, slot).wait()

        @pl.when(i >= _K)
        def _():
            out_copy(i - _K, slot).wait()

        out_buf[slot] = in_buf[slot] * scale[None, None, :]

        out_copy(i, slot).start()

        @pl.when(i + _K < n)
        def _():
            in_copy(i + _K, slot).start()

        return carry

    jax.lax.fori_loop(0, n, step, 0)

    for k in range(_K):
        out_copy(n - _K + k, (n - _K + k) % _K).wait()


def kernel(x, stage, embeddings):
    b, c, h, w = x.shape
    xt = jnp.transpose(x, (0, 2, 3, 1)).reshape(b, h * w, c)
    stage_arr = jnp.asarray(stage, jnp.int32).reshape((1,))

    out = pl.pallas_call(
        _scale_body,
        in_specs=[
            pl.BlockSpec(memory_space=pltpu.SMEM),
            pl.BlockSpec(memory_space=pltpu.VMEM),
            pl.BlockSpec(memory_space=pltpu.HBM),
        ],
        out_specs=pl.BlockSpec(memory_space=pltpu.HBM),
        out_shape=jax.ShapeDtypeStruct((b, h * w, c), x.dtype),
        scratch_shapes=[
            pltpu.VMEM((_K, _C, h * w, c), x.dtype),
            pltpu.VMEM((_K, _C, h * w, c), x.dtype),
            pltpu.SemaphoreType.DMA((_K,)),
            pltpu.SemaphoreType.DMA((_K,)),
        ],
    )(stage_arr, embeddings, xt)
    return out.reshape(b, h, w, c).transpose(0, 3, 1, 2)


# speedup vs baseline: 1.0012x; 1.0012x over previous
import jax
import jax.numpy as jnp
from jax.experimental import pallas as pl
from jax.experimental.pallas import tpu as pltpu

_C = 16
_K = 2


def _scale_body(stage_ref, emb_ref, x_hbm, o_hbm, buf, in_sem, out_sem):
    n = x_hbm.shape[0] // _C
    s = stage_ref[0]
    emb = emb_ref[...]
    row = jax.lax.broadcasted_iota(jnp.int32, emb.shape, 0)
    scale = jnp.sum(jnp.where(row == s, emb, 0.0), axis=0)

    def in_copy(i, slot):
        return pltpu.make_async_copy(x_hbm.at[pl.ds(i * _C, _C)],
                                     buf.at[slot], in_sem.at[slot])

    def out_copy(i, slot):
        return pltpu.make_async_copy(buf.at[slot],
                                     o_hbm.at[pl.ds(i * _C, _C)],
                                     out_sem.at[slot])

    for k in range(_K):
        in_copy(k, k).start()

    def step(i, carry):
        slot = jax.lax.rem(i, _K)
        in_copy(i, slot).wait()
        buf[slot] = buf[slot] * scale[None, None, :]
        out_copy(i, slot).start()

        @pl.when(i + _K < n)
        def _():
            out_copy(i, slot).wait()
            in_copy(i + _K, slot).start()

        return carry

    jax.lax.fori_loop(0, n, step, 0)

    for k in range(_K):
        out_copy(n - _K + k, (n - _K + k) % _K).wait()


def kernel(x, stage, embeddings):
    b, c, h, w = x.shape
    xt = jnp.transpose(x, (0, 2, 3, 1)).reshape(b, h * w, c)
    stage_arr = jnp.asarray(stage, jnp.int32).reshape((1,))

    out = pl.pallas_call(
        _scale_body,
        in_specs=[
            pl.BlockSpec(memory_space=pltpu.SMEM),
            pl.BlockSpec(memory_space=pltpu.VMEM),
            pl.BlockSpec(memory_space=pltpu.HBM),
        ],
        out_specs=pl.BlockSpec(memory_space=pltpu.HBM),
        out_shape=jax.ShapeDtypeStruct((b, h * w, c), x.dtype),
        scratch_shapes=[
            pltpu.VMEM((_K, _C, h * w, c), x.dtype),
            pltpu.SemaphoreType.DMA((_K,)),
            pltpu.SemaphoreType.DMA((_K,)),
        ],
    )(stage_arr, embeddings, xt)
    return out.reshape(b, h, w, c).transpose(0, 3, 1, 2)


# speedup vs baseline: 1.0190x; 1.0178x over previous
import jax
import jax.numpy as jnp
from jax.experimental import pallas as pl
from jax.experimental.pallas import tpu as pltpu

_B = 8


def _scale_kernel(stage_ref, emb_ref, x_ref, o_ref):
    s = stage_ref[0]
    emb = emb_ref[...]
    row = jax.lax.broadcasted_iota(jnp.int32, emb.shape, 0)
    scale = jnp.sum(jnp.where(row == s, emb, 0.0), axis=0)
    o_ref[...] = x_ref[...] * scale[None, None, :]


def kernel(x, stage, embeddings):
    b, c, h, w = x.shape
    xt = jnp.transpose(x, (0, 2, 3, 1)).reshape(b, h * w, c)
    stage_arr = jnp.asarray(stage, jnp.int32).reshape((1,))

    out = pl.pallas_call(
        _scale_kernel,
        grid=(b // _B,),
        in_specs=[
            pl.BlockSpec(memory_space=pltpu.SMEM),
            pl.BlockSpec(embeddings.shape, lambda i: (0, 0)),
            pl.BlockSpec((_B, h * w, c), lambda i: (i, 0, 0)),
        ],
        out_specs=pl.BlockSpec((_B, h * w, c), lambda i: (i, 0, 0)),
        out_shape=jax.ShapeDtypeStruct((b, h * w, c), x.dtype),
        compiler_params=pltpu.CompilerParams(
            dimension_semantics=("parallel",),
        ),
    )(stage_arr, embeddings, xt)
    return out.reshape(b, h, w, c).transpose(0, 3, 1, 2)
